# Initial kernel scaffold; baseline (speedup 1.0000x reference)
import jax
import jax.numpy as jnp
from jax.experimental import pallas as pl


def kernel(edge_index, user_weight, item_weight):
    raise NotImplementedError("write your pallas kernel here")


# speedup vs baseline: 6.9223x; 6.9223x over previous
import functools

import jax
import jax.numpy as jnp
from jax import lax
from jax.experimental import pallas as pl
from jax.experimental.pallas import tpu as pltpu
from jax.experimental.pallas import tpu_sc as plsc

N_USERS = 25000
N_ITEMS = 25000
N = N_USERS + N_ITEMS
EMB = 64
N_LAYERS = 3
E = 800000

NC = 2
NS = 16
L = 16

HALF = N // NC
PHALF = 25088
NPAD = NC * PHALF
PAD_SHIFT = PHALF - HALF

EC = 400
EPT = E // NS
ECHUNKS = EPT // EC

ZR = 224
ZPT = PHALF // NS // ZR

QHALF = PHALF // 2
QPT = QHALF // NS
NRB = 112

_MESH = plsc.VectorSubcoreMesh(core_axis_name="c", subcore_axis_name="s")
_PARAMS = pltpu.CompilerParams(use_tc_tiling_on_sc=False)


def _rsqrt_newton(x):
  i = lax.bitcast_convert_type(x, jnp.int32)
  i = 0x5F3759DF - (i >> 1)
  y = lax.bitcast_convert_type(i, jnp.float32)
  for _ in range(3):
    y = y * (1.5 - 0.5 * x * y * y)
  return y


def _zero_vmem(ref, rows, cols):
  zv = jnp.zeros((L,), jnp.float32)

  def body(r, _):
    for cc in range(cols // L):
      ref[r, pl.ds(cc * L, L)] = zv
    return _

  lax.fori_loop(0, rows, body, None)


def _deg_dis_y0_body(col_hbm, x0_hbm, dis_hbm, y0_hbm,
                     cbuf, clbuf, ones_v, dbuf, disb, xbuf, disn, deg_sh):
  c = lax.axis_index("c")
  s = lax.axis_index("s")
  base = c * HALF

  one = jnp.full((L,), 1.0, jnp.float32)

  def fill(r, _):
    ones_v[r, :] = one
    return _

  lax.fori_loop(0, EC, fill, None)

  iota16 = lax.iota(jnp.int32, L)
  onehots = [jnp.where(iota16 == k, 1.0, 0.0).astype(jnp.float32)
             for k in range(L)]

  for p in range(2):
    qbase = base + p * QHALF

    _zero_vmem(dbuf, QPT, L)
    pltpu.sync_copy(dbuf, deg_sh.at[pl.ds(s * QPT, QPT)])
    plsc.subcore_barrier()

    def echunk(j, _):
      e0 = s * EPT + j * EC
      pltpu.sync_copy(col_hbm.at[pl.ds(e0, EC)], cbuf)

      def grp(g, _):
        v = cbuf[pl.ds(g * L, L)]
        inr = (v >= qbase) & (v < qbase + QHALF) & (v < base + HALF)
        clbuf[pl.ds(g * L, L)] = jnp.where(inr, v - qbase, -1)
        return _

      lax.fori_loop(0, EC // L, grp, None)
      pltpu.sync_copy(ones_v, deg_sh.at[plsc.Indices(clbuf, ignored_value=-1)],
                      add=True)
      return _

    lax.fori_loop(0, ECHUNKS, echunk, None)
    plsc.subcore_barrier()

    pltpu.sync_copy(deg_sh.at[pl.ds(s * QPT, QPT)], dbuf)

    def dgrp(g, _):
      deg = dbuf[g * L, :] * onehots[0]
      for k in range(1, L):
        deg = deg + dbuf[g * L + k, :] * onehots[k]
      y = _rsqrt_newton(deg)
      disb[pl.ds(g * L, L)] = jnp.where(deg > 0.0, y, 0.0)
      return _

    lax.fori_loop(0, QPT // L, dgrp, None)
    pltpu.sync_copy(disb, dis_hbm.at[pl.ds(c * PHALF + p * QHALF + s * QPT,
                                           QPT)])
    plsc.subcore_barrier()

  def nchunk(j, _):
    n0 = c * PHALF + (s * ZPT + j) * ZR
    pltpu.sync_copy(x0_hbm.at[pl.ds(n0, ZR)], xbuf)
    pltpu.sync_copy(dis_hbm.at[pl.ds(n0, ZR)], disn)

    def rowgrp(g, _):
      dv = disn[pl.ds(g * L, L)]
      for k in range(L):
        d = dv[k]
        r = g * L + k
        for cc in range(EMB // L):
          xbuf[r, pl.ds(cc * L, L)] = xbuf[r, pl.ds(cc * L, L)] * d
      return _

    lax.fori_loop(0, ZR // L, rowgrp, None)
    pltpu.sync_copy(xbuf, y0_hbm.at[pl.ds(n0, ZR)])
    return _

  lax.fori_loop(0, ZPT, nchunk, None)


def _layer_body(row_hbm, col_hbm, y_hbm, dis_hbm, sp_hbm, s_hbm, ynext_hbm,
                rbuf, cbuf, rpbuf, clbuf, gbuf, abuf, sbuf, disn, acc_sh, sem):
  c = lax.axis_index("c")
  s = lax.axis_index("s")
  base = c * HALF

  for p in range(2):
    qbase = base + p * QHALF

    _zero_vmem(abuf, NRB, EMB)

    def zchunk(j, _):
      pltpu.sync_copy(abuf, acc_sh.at[pl.ds(s * QPT + j * NRB, NRB)])
      return _

    lax.fori_loop(0, QPT // NRB, zchunk, None)
    plsc.subcore_barrier()

    def echunk(j, _):
      e0 = s * EPT + j * EC
      pltpu.sync_copy(row_hbm.at[pl.ds(e0, EC)], rbuf)
      pltpu.sync_copy(col_hbm.at[pl.ds(e0, EC)], cbuf)

      def grp(g, _):
        v = cbuf[pl.ds(g * L, L)]
        inr = (v >= qbase) & (v < qbase + QHALF) & (v < base + HALF)
        clbuf[pl.ds(g * L, L)] = jnp.where(inr, v - qbase, -1)
        r = rbuf[pl.ds(g * L, L)]
        rp = jnp.where(r >= HALF, r + PAD_SHIFT, r)
        rpbuf[pl.ds(g * L, L)] = jnp.where(inr, rp, -1)
        return _

      lax.fori_loop(0, EC // L, grp, None)
      pltpu.async_copy(y_hbm.at[plsc.Indices(rpbuf, ignored_value=-1)],
                       gbuf, sem).wait()
      pltpu.sync_copy(gbuf, acc_sh.at[plsc.Indices(clbuf, ignored_value=-1)],
                      add=True)
      return _

    lax.fori_loop(0, ECHUNKS, echunk, None)
    plsc.subcore_barrier()

    def nchunk(j, _):
      nl = s * QPT + j * NRB
      n0 = c * PHALF + p * QHALF + nl
      pltpu.sync_copy(acc_sh.at[pl.ds(nl, NRB)], abuf)
      pltpu.sync_copy(dis_hbm.at[pl.ds(n0, NRB)], disn)
      pltpu.sync_copy(sp_hbm.at[pl.ds(n0, NRB)], sbuf)

      def rowgrp(g, _):
        dv = disn[pl.ds(g * L, L)]
        for k in range(L):
          d = dv[k]
          r = g * L + k
          for cc in range(EMB // L):
            x = abuf[r, pl.ds(cc * L, L)] * d
            sbuf[r, pl.ds(cc * L, L)] = sbuf[r, pl.ds(cc * L, L)] + x
            abuf[r, pl.ds(cc * L, L)] = x * d
        return _

      lax.fori_loop(0, NRB // L, rowgrp, None)
      pltpu.sync_copy(sbuf, s_hbm.at[pl.ds(n0, NRB)])
      pltpu.sync_copy(abuf, ynext_hbm.at[pl.ds(n0, NRB)])
      return _

    lax.fori_loop(0, QPT // NRB, nchunk, None)
    plsc.subcore_barrier()


def _deg_dis_y0(col_arr, x0p):
  return pl.kernel(
      _deg_dis_y0_body,
      out_type=(
          jax.ShapeDtypeStruct((NPAD,), jnp.float32),
          jax.ShapeDtypeStruct((NPAD, EMB), jnp.float32),
      ),
      mesh=_MESH,
      scratch_types=[
          pltpu.VMEM((EC,), jnp.int32),
          pltpu.VMEM((EC,), jnp.int32),
          pltpu.VMEM((EC, L), jnp.float32),
          pltpu.VMEM((QPT, L), jnp.float32),
          pltpu.VMEM((QPT,), jnp.float32),
          pltpu.VMEM((ZR, EMB), jnp.float32),
          pltpu.VMEM((ZR,), jnp.float32),
          pltpu.VMEM_SHARED((QHALF, L), jnp.float32),
      ],
      compiler_params=_PARAMS,
  )(col_arr, x0p)


def _layer(row_arr, col_arr, y_prev, dis_pad, s_prev):
  return pl.kernel(
      _layer_body,
      out_type=(
          jax.ShapeDtypeStruct((NPAD, EMB), jnp.float32),
          jax.ShapeDtypeStruct((NPAD, EMB), jnp.float32),
      ),
      mesh=_MESH,
      scratch_types=[
          pltpu.VMEM((EC,), jnp.int32),
          pltpu.VMEM((EC,), jnp.int32),
          pltpu.VMEM((EC,), jnp.int32),
          pltpu.VMEM((EC,), jnp.int32),
          pltpu.VMEM((EC, EMB), jnp.float32),
          pltpu.VMEM((NRB, EMB), jnp.float32),
          pltpu.VMEM((NRB, EMB), jnp.float32),
          pltpu.VMEM((NRB,), jnp.float32),
          pltpu.VMEM_SHARED((QHALF, EMB), jnp.float32),
          pltpu.SemaphoreType.DMA,
      ],
      compiler_params=_PARAMS,
  )(row_arr, col_arr, y_prev, dis_pad, s_prev)


def kernel(edge_index, user_weight, item_weight):
  row_arr = edge_index[0]
  col_arr = edge_index[1]
  x0 = jnp.concatenate([user_weight, item_weight], axis=0)
  x0p = jnp.zeros((NPAD, EMB), jnp.float32)
  x0p = x0p.at[:HALF].set(x0[:HALF]).at[PHALF:PHALF + HALF].set(x0[HALF:])

  dis_pad, y = _deg_dis_y0(col_arr, x0p)
  s = x0p
  for _ in range(N_LAYERS):
    s, y = _layer(row_arr, col_arr, y, dis_pad, s)

  final = jnp.concatenate([s[:HALF], s[PHALF:PHALF + HALF]], axis=0)
  final = final * (1.0 / (N_LAYERS + 1))
  return (final[:N_USERS], final[N_USERS:])


# speedup vs baseline: 12.1882x; 1.7607x over previous
import functools

import jax
import jax.numpy as jnp
from jax import lax
from jax.experimental import pallas as pl
from jax.experimental.pallas import tpu as pltpu
from jax.experimental.pallas import tpu_sc as plsc

N_USERS = 25000
N_ITEMS = 25000
N = N_USERS + N_ITEMS
EMB = 64
N_LAYERS = 3
E = 800000

NC = 2
NS = 16
L = 16

HALF = N // NC
PHALF = 25088
NPAD = NC * PHALF
PAD_SHIFT = PHALF - HALF

EC = 400
EPT = E // NS
ECHUNKS = EPT // EC

ZR = 224
ZPT = PHALF // NS // ZR

QHALF = PHALF // 2
QPT = QHALF // NS
NRB = 112

_MESH = plsc.VectorSubcoreMesh(core_axis_name="c", subcore_axis_name="s")
_PARAMS = pltpu.CompilerParams(use_tc_tiling_on_sc=False)


def _rsqrt_newton(x):
  i = lax.bitcast_convert_type(x, jnp.int32)
  i = 0x5F3759DF - (i >> 1)
  y = lax.bitcast_convert_type(i, jnp.float32)
  for _ in range(3):
    y = y * (1.5 - 0.5 * x * y * y)
  return y


def _zero_vmem(ref, rows, cols):
  zv = jnp.zeros((L,), jnp.float32)

  def body(r, _):
    for cc in range(cols // L):
      ref[r, pl.ds(cc * L, L)] = zv
    return _

  lax.fori_loop(0, rows, body, None)


def _deg_dis_y0_body(col_hbm, x0_hbm, dis_hbm, y0_hbm,
                     cbuf, clbuf, ones_v, dbuf, disb, xbuf, disn, deg_sh):
  c = lax.axis_index("c")
  s = lax.axis_index("s")
  base = c * HALF

  one = jnp.full((L,), 1.0, jnp.float32)

  def fill(r, _):
    ones_v[r, :] = one
    return _

  lax.fori_loop(0, EC, fill, None)

  iota16 = lax.iota(jnp.int32, L)
  onehots = [jnp.where(iota16 == k, 1.0, 0.0).astype(jnp.float32)
             for k in range(L)]

  for p in range(2):
    qbase = base + p * QHALF

    _zero_vmem(dbuf, QPT, L)
    pltpu.sync_copy(dbuf, deg_sh.at[pl.ds(s * QPT, QPT)])
    plsc.subcore_barrier()

    def echunk(j, _):
      e0 = s * EPT + j * EC
      pltpu.sync_copy(col_hbm.at[pl.ds(e0, EC)], cbuf)

      def grp(g, _):
        v = cbuf[pl.ds(g * L, L)]
        inr = (v >= qbase) & (v < qbase + QHALF) & (v < base + HALF)
        clbuf[pl.ds(g * L, L)] = jnp.where(inr, v - qbase, -1)
        return _

      lax.fori_loop(0, EC // L, grp, None)
      pltpu.sync_copy(ones_v, deg_sh.at[plsc.Indices(clbuf, ignored_value=-1)],
                      add=True)
      return _

    lax.fori_loop(0, ECHUNKS, echunk, None)
    plsc.subcore_barrier()

    pltpu.sync_copy(deg_sh.at[pl.ds(s * QPT, QPT)], dbuf)

    def dgrp(g, _):
      deg = dbuf[g * L, :] * onehots[0]
      for k in range(1, L):
        deg = deg + dbuf[g * L + k, :] * onehots[k]
      y = _rsqrt_newton(deg)
      disb[pl.ds(g * L, L)] = jnp.where(deg > 0.0, y, 0.0)
      return _

    lax.fori_loop(0, QPT // L, dgrp, None)
    pltpu.sync_copy(disb, dis_hbm.at[pl.ds(c * PHALF + p * QHALF + s * QPT,
                                           QPT)])
    plsc.subcore_barrier()

  def nchunk(j, _):
    n0 = c * PHALF + (s * ZPT + j) * ZR
    pltpu.sync_copy(x0_hbm.at[pl.ds(n0, ZR)], xbuf)
    pltpu.sync_copy(dis_hbm.at[pl.ds(n0, ZR)], disn)

    def rowgrp(g, _):
      dv = disn[pl.ds(g * L, L)]
      for k in range(L):
        d = dv[k]
        r = g * L + k
        for cc in range(EMB // L):
          xbuf[r, pl.ds(cc * L, L)] = xbuf[r, pl.ds(cc * L, L)] * d
      return _

    lax.fori_loop(0, ZR // L, rowgrp, None)
    pltpu.sync_copy(xbuf, y0_hbm.at[pl.ds(n0, ZR)])
    return _

  lax.fori_loop(0, ZPT, nchunk, None)


def _layer_body(row_hbm, col_hbm, y_hbm, dis_hbm, sp_hbm, s_hbm, ynext_hbm,
                rbuf, cbuf, rpbuf, clbuf, gbuf, abuf, sbuf, disn, acc_sh,
                semg0, semg1, semi):
  c = lax.axis_index("c")
  s = lax.axis_index("s")
  base = c * HALF
  semg = (semg0, semg1)

  for p in range(2):
    qbase = base + p * QHALF

    _zero_vmem(abuf, NRB, EMB)

    def zchunk(j, _):
      pltpu.sync_copy(abuf, acc_sh.at[pl.ds(s * QPT + j * NRB, NRB)])
      return _

    lax.fori_loop(0, QPT // NRB, zchunk, None)
    plsc.subcore_barrier()

    def prep_masks(slot):
      def grp(g, _):
        v = cbuf[slot, pl.ds(g * L, L)]
        inr = (v >= qbase) & (v < qbase + QHALF) & (v < base + HALF)
        clbuf[slot, pl.ds(g * L, L)] = jnp.where(inr, v - qbase, -1)
        r = rbuf[slot, pl.ds(g * L, L)]
        rp = jnp.where(r >= HALF, r + PAD_SHIFT, r)
        rpbuf[slot, pl.ds(g * L, L)] = jnp.where(inr, rp, -1)
        return _

      lax.fori_loop(0, EC // L, grp, None)

    def start_gather(slot):
      pltpu.async_copy(
          y_hbm.at[plsc.Indices(rpbuf.at[slot], ignored_value=-1)],
          gbuf.at[slot], semg[slot])

    def wait_gather(slot):
      pltpu.make_async_copy(
          y_hbm.at[plsc.Indices(rpbuf.at[slot], ignored_value=-1)],
          gbuf.at[slot], semg[slot]).wait()

    def scatter(slot):
      pltpu.sync_copy(
          gbuf.at[slot],
          acc_sh.at[plsc.Indices(clbuf.at[slot], ignored_value=-1)],
          add=True)

    e00 = s * EPT
    pltpu.sync_copy(row_hbm.at[pl.ds(e00, EC)], rbuf.at[0])
    pltpu.sync_copy(col_hbm.at[pl.ds(e00, EC)], cbuf.at[0])
    prep_masks(0)
    start_gather(0)

    def pair(jj, _):
      for slot in (0, 1):
        j = 2 * jj + slot
        other = 1 - slot
        e1 = s * EPT + (j + 1) * EC
        dr = pltpu.async_copy(row_hbm.at[pl.ds(e1, EC)], rbuf.at[other], semi)
        dc = pltpu.async_copy(col_hbm.at[pl.ds(e1, EC)], cbuf.at[other], semi)
        wait_gather(slot)
        dr.wait()
        dc.wait()
        prep_masks(other)
        start_gather(other)
        scatter(slot)
      return _

    lax.fori_loop(0, ECHUNKS // 2, pair, None)
    wait_gather(0)
    scatter(0)
    plsc.subcore_barrier()

    def nchunk(j, _):
      nl = s * QPT + j * NRB
      n0 = c * PHALF + p * QHALF + nl
      pltpu.sync_copy(acc_sh.at[pl.ds(nl, NRB)], abuf)
      pltpu.sync_copy(dis_hbm.at[pl.ds(n0, NRB)], disn)
      pltpu.sync_copy(sp_hbm.at[pl.ds(n0, NRB)], sbuf)

      def rowgrp(g, _):
        dv = disn[pl.ds(g * L, L)]
        for k in range(L):
          d = dv[k]
          r = g * L + k
          for cc in range(EMB // L):
            x = abuf[r, pl.ds(cc * L, L)] * d
            sbuf[r, pl.ds(cc * L, L)] = sbuf[r, pl.ds(cc * L, L)] + x
            abuf[r, pl.ds(cc * L, L)] = x * d
        return _

      lax.fori_loop(0, NRB // L, rowgrp, None)
      pltpu.sync_copy(sbuf, s_hbm.at[pl.ds(n0, NRB)])
      pltpu.sync_copy(abuf, ynext_hbm.at[pl.ds(n0, NRB)])
      return _

    lax.fori_loop(0, QPT // NRB, nchunk, None)
    plsc.subcore_barrier()


def _deg_dis_y0(col_arr, x0p):
  return pl.kernel(
      _deg_dis_y0_body,
      out_type=(
          jax.ShapeDtypeStruct((NPAD,), jnp.float32),
          jax.ShapeDtypeStruct((NPAD, EMB), jnp.float32),
      ),
      mesh=_MESH,
      scratch_types=[
          pltpu.VMEM((EC,), jnp.int32),
          pltpu.VMEM((EC,), jnp.int32),
          pltpu.VMEM((EC, L), jnp.float32),
          pltpu.VMEM((QPT, L), jnp.float32),
          pltpu.VMEM((QPT,), jnp.float32),
          pltpu.VMEM((ZR, EMB), jnp.float32),
          pltpu.VMEM((ZR,), jnp.float32),
          pltpu.VMEM_SHARED((QHALF, L), jnp.float32),
      ],
      compiler_params=_PARAMS,
  )(col_arr, x0p)


def _layer(row_arr, col_arr, y_prev, dis_pad, s_prev):
  return pl.kernel(
      _layer_body,
      out_type=(
          jax.ShapeDtypeStruct((NPAD, EMB), jnp.float32),
          jax.ShapeDtypeStruct((NPAD, EMB), jnp.float32),
      ),
      mesh=_MESH,
      scratch_types=[
          pltpu.VMEM((2, EC), jnp.int32),
          pltpu.VMEM((2, EC), jnp.int32),
          pltpu.VMEM((2, EC), jnp.int32),
          pltpu.VMEM((2, EC), jnp.int32),
          pltpu.VMEM((2, EC, EMB), jnp.float32),
          pltpu.VMEM((NRB, EMB), jnp.float32),
          pltpu.VMEM((NRB, EMB), jnp.float32),
          pltpu.VMEM((NRB,), jnp.float32),
          pltpu.VMEM_SHARED((QHALF, EMB), jnp.float32),
          pltpu.SemaphoreType.DMA,
          pltpu.SemaphoreType.DMA,
          pltpu.SemaphoreType.DMA,
      ],
      compiler_params=_PARAMS,
  )(row_arr, col_arr, y_prev, dis_pad, s_prev)


def kernel(edge_index, user_weight, item_weight):
  row_arr = edge_index[0]
  col_arr = edge_index[1]
  x0 = jnp.concatenate([user_weight, item_weight], axis=0)
  x0p = jnp.zeros((NPAD, EMB), jnp.float32)
  x0p = x0p.at[:HALF].set(x0[:HALF]).at[PHALF:PHALF + HALF].set(x0[HALF:])

  dis_pad, y = _deg_dis_y0(col_arr, x0p)
  s = x0p
  for _ in range(N_LAYERS):
    s, y = _layer(row_arr, col_arr, y, dis_pad, s)

  final = jnp.concatenate([s[:HALF], s[PHALF:PHALF + HALF]], axis=0)
  final = final * (1.0 / (N_LAYERS + 1))
  return (final[:N_USERS], final[N_USERS:])


# speedup vs baseline: 12.6751x; 1.0399x over previous
import jax
import jax.numpy as jnp
from jax import lax
from jax.experimental import pallas as pl
from jax.experimental.pallas import tpu as pltpu
from jax.experimental.pallas import tpu_sc as plsc

N_USERS = 25000
N_ITEMS = 25000
N = N_USERS + N_ITEMS
EMB = 64
N_LAYERS = 3
E = 800000

NC = 2
NS = 16
L = 16

HALF = N // NC
PHALF = 25088
NPAD = NC * PHALF
PAD_SHIFT = PHALF - HALF

EC = 400
EPT = E // NS
ECHUNKS = EPT // EC

ZR = 224
ZPT = PHALF // NS // ZR
HPT = PHALF // NS

QHALF = PHALF // 2
QPT = QHALF // NS
NRB = 112

_MESH = plsc.VectorSubcoreMesh(core_axis_name="c", subcore_axis_name="s")
_PARAMS = pltpu.CompilerParams(use_tc_tiling_on_sc=False)


def _rsqrt_newton(x):
  i = lax.bitcast_convert_type(x, jnp.int32)
  i = 0x5F3759DF - (i >> 1)
  y = lax.bitcast_convert_type(i, jnp.float32)
  for _ in range(3):
    y = y * (1.5 - 0.5 * x * y * y)
  return y


def _zero_vmem(ref, rows, cols):
  zv = jnp.zeros((L,), jnp.float32)

  def body(r, _):
    for cc in range(cols // L):
      ref[r, pl.ds(cc * L, L)] = zv
    return _

  lax.fori_loop(0, rows, body, None)


def _deg_dis_y0_body(col_hbm, x0_hbm, dis_hbm, y0_hbm,
                     cbuf, dgbuf, ones_v, dbuf, disb, xbuf, disn,
                     deg_sh, semi):
  c = lax.axis_index("c")
  s = lax.axis_index("s")
  base = c * HALF

  one = jnp.full((L,), 1.0, jnp.float32)

  def fill(r, _):
    ones_v[r, :] = one
    return _

  lax.fori_loop(0, EC, fill, None)

  iota16 = lax.iota(jnp.int32, L)
  onehots = [jnp.where(iota16 == k, 1.0, 0.0).astype(jnp.float32)
             for k in range(L)]

  for q in range(2):
    qbase = base + q * QHALF

    _zero_vmem(dbuf, QPT, L)
    pltpu.sync_copy(dbuf, deg_sh.at[pl.ds(s * QPT, QPT)])
    plsc.subcore_barrier()

    def masks(other):
      def grp(g, _):
        v = cbuf[other, pl.ds(g * L, L)]
        inr = (v >= qbase) & (v < qbase + QHALF) & (v < base + HALF)
        dgbuf[other, pl.ds(g * L, L)] = jnp.where(inr, v - qbase, -1)
        return _

      lax.fori_loop(0, EC // L, grp, None)

    pltpu.sync_copy(col_hbm.at[pl.ds(s * EPT, EC)], cbuf.at[0])
    masks(0)

    def echunk_pair(jj, _):
      for slot in (0, 1):
        j = 2 * jj + slot
        other = 1 - slot
        e1 = s * EPT + (j + 1) * EC
        dc = pltpu.async_copy(col_hbm.at[pl.ds(e1, EC)], cbuf.at[other],
                              semi)
        pltpu.sync_copy(
            ones_v,
            deg_sh.at[plsc.Indices(dgbuf.at[slot], ignored_value=-1)],
            add=True)
        dc.wait()
        masks(other)
      return _

    lax.fori_loop(0, ECHUNKS // 2, echunk_pair, None)
    pltpu.sync_copy(
        ones_v,
        deg_sh.at[plsc.Indices(dgbuf.at[0], ignored_value=-1)],
        add=True)
    plsc.subcore_barrier()

    pltpu.sync_copy(deg_sh.at[pl.ds(s * QPT, QPT)], dbuf)

    def dgrp(g, _):
      deg = dbuf[g * L, :] * onehots[0]
      for k in range(1, L):
        deg = deg + dbuf[g * L + k, :] * onehots[k]
      y = _rsqrt_newton(deg)
      disb[pl.ds(g * L, L)] = jnp.where(deg > 0.0, y, 0.0)
      return _

    lax.fori_loop(0, QPT // L, dgrp, None)
    pltpu.sync_copy(disb, dis_hbm.at[pl.ds(c * PHALF + q * QHALF + s * QPT,
                                           QPT)])
    plsc.subcore_barrier()

  def nchunk(j, _):
    n0 = c * PHALF + (s * ZPT + j) * ZR
    pltpu.sync_copy(x0_hbm.at[pl.ds(n0, ZR)], xbuf)
    pltpu.sync_copy(dis_hbm.at[pl.ds(n0, ZR)], disn)

    def rowgrp(g, _):
      dv = disn[pl.ds(g * L, L)]
      for k in range(L):
        d = dv[k]
        r = g * L + k
        for cc in range(EMB // L):
          xbuf[r, pl.ds(cc * L, L)] = xbuf[r, pl.ds(cc * L, L)] * d
      return _

    lax.fori_loop(0, ZR // L, rowgrp, None)
    pltpu.sync_copy(xbuf, y0_hbm.at[pl.ds(n0, ZR)])
    return _

  lax.fori_loop(0, ZPT, nchunk, None)


def _layer_body(row_hbm, col_hbm, ysrc, dis_hbm, sp_hbm,
                s_hbm, ynext_hbm,
                rbuf, cbuf, mbuf, gbuf, abuf, sbuf, disn, acc_sh,
                semg0, semg1, semi):
  c = lax.axis_index("c")
  s = lax.axis_index("s")
  base = c * HALF
  semg = (semg0, semg1)

  _zero_vmem(abuf, NRB, EMB)

  for p in range(2):
    qbase = base + p * QHALF

    def zchunk(j, _):
      pltpu.sync_copy(abuf, acc_sh.at[pl.ds(s * QPT + j * NRB, NRB)])
      return _

    lax.fori_loop(0, QPT // NRB, zchunk, None)
    plsc.subcore_barrier()

    def masks(slot):
      def grp(g, _):
        v = cbuf[slot, pl.ds(g * L, L)]
        r = rbuf[slot, pl.ds(g * L, L)]
        rp = jnp.where(r >= HALF, r + PAD_SHIFT, r)
        inr = (v >= qbase) & (v < qbase + QHALF) & (v < base + HALF)
        mbuf[slot, 0, pl.ds(g * L, L)] = jnp.where(inr, rp, -1)
        mbuf[slot, 1, pl.ds(g * L, L)] = jnp.where(inr, v - qbase, -1)
        return _

      lax.fori_loop(0, EC // L, grp, None)

    def start_gather(slot):
      pltpu.async_copy(
          ysrc.at[plsc.Indices(mbuf.at[slot, 0], ignored_value=-1)],
          gbuf.at[slot], semg[slot])

    def wait_gather(slot):
      pltpu.make_async_copy(
          ysrc.at[plsc.Indices(mbuf.at[slot, 0], ignored_value=-1)],
          gbuf.at[slot], semg[slot]).wait()

    def scatter(slot):
      pltpu.sync_copy(
          gbuf.at[slot],
          acc_sh.at[plsc.Indices(mbuf.at[slot, 1], ignored_value=-1)],
          add=True)

    e00 = s * EPT
    pltpu.sync_copy(row_hbm.at[pl.ds(e00, EC)], rbuf.at[0])
    pltpu.sync_copy(col_hbm.at[pl.ds(e00, EC)], cbuf.at[0])
    masks(0)
    start_gather(0)

    def pair(jj, _):
      for slot in (0, 1):
        j = 2 * jj + slot
        other = 1 - slot
        e1 = s * EPT + (j + 1) * EC
        dr = pltpu.async_copy(row_hbm.at[pl.ds(e1, EC)], rbuf.at[other],
                              semi)
        dc = pltpu.async_copy(col_hbm.at[pl.ds(e1, EC)], cbuf.at[other],
                              semi)
        wait_gather(slot)
        dr.wait()
        dc.wait()
        masks(other)
        start_gather(other)
        scatter(slot)
      return _

    lax.fori_loop(0, ECHUNKS // 2, pair, None)
    wait_gather(0)
    scatter(0)
    plsc.subcore_barrier()

    def nchunk(j, _):
      nl = s * QPT + j * NRB
      n0 = c * PHALF + p * QHALF + nl
      pltpu.sync_copy(acc_sh.at[pl.ds(nl, NRB)], abuf)
      pltpu.sync_copy(dis_hbm.at[pl.ds(n0, NRB)], disn)
      pltpu.sync_copy(sp_hbm.at[pl.ds(n0, NRB)], sbuf)

      def rowgrp(g, _):
        dv = disn[pl.ds(g * L, L)]
        for k in range(L):
          d = dv[k]
          r = g * L + k
          for cc in range(EMB // L):
            x = abuf[r, pl.ds(cc * L, L)] * d
            sbuf[r, pl.ds(cc * L, L)] = sbuf[r, pl.ds(cc * L, L)] + x
            abuf[r, pl.ds(cc * L, L)] = x * d
        return _

      lax.fori_loop(0, NRB // L, rowgrp, None)
      pltpu.sync_copy(sbuf, s_hbm.at[pl.ds(n0, NRB)])
      pltpu.sync_copy(abuf, ynext_hbm.at[pl.ds(n0, NRB)])
      return _

    lax.fori_loop(0, QPT // NRB, nchunk, None)

    if p == 0:
      _zero_vmem(abuf, NRB, EMB)
      plsc.subcore_barrier()


def _deg_dis_y0(col_arr, x0p):
  return pl.kernel(
      _deg_dis_y0_body,
      out_type=(
          jax.ShapeDtypeStruct((NPAD,), jnp.float32),
          jax.ShapeDtypeStruct((NPAD, EMB), jnp.float32),
      ),
      mesh=_MESH,
      scratch_types=[
          pltpu.VMEM((2, EC), jnp.int32),
          pltpu.VMEM((2, EC), jnp.int32),
          pltpu.VMEM((EC, L), jnp.float32),
          pltpu.VMEM((QPT, L), jnp.float32),
          pltpu.VMEM((QPT,), jnp.float32),
          pltpu.VMEM((ZR, EMB), jnp.float32),
          pltpu.VMEM((ZR,), jnp.float32),
          pltpu.VMEM_SHARED((QHALF, L), jnp.float32),
          pltpu.SemaphoreType.DMA,
      ],
      compiler_params=_PARAMS,
  )(col_arr, x0p)


def _layer(row_arr, col_arr, y_prev, dis_pad, s_prev):
  return pl.kernel(
      _layer_body,
      out_type=(
          jax.ShapeDtypeStruct((NPAD, EMB), jnp.float32),
          jax.ShapeDtypeStruct((NPAD, EMB), jnp.float32),
      ),
      mesh=_MESH,
      scratch_types=[
          pltpu.VMEM((2, EC), jnp.int32),
          pltpu.VMEM((2, EC), jnp.int32),
          pltpu.VMEM((2, 2, EC), jnp.int32),
          pltpu.VMEM((2, EC, EMB), jnp.float32),
          pltpu.VMEM((NRB, EMB), jnp.float32),
          pltpu.VMEM((NRB, EMB), jnp.float32),
          pltpu.VMEM((NRB,), jnp.float32),
          pltpu.VMEM_SHARED((QHALF, EMB), jnp.float32),
          pltpu.SemaphoreType.DMA,
          pltpu.SemaphoreType.DMA,
          pltpu.SemaphoreType.DMA,
      ],
      compiler_params=_PARAMS,
  )(row_arr, col_arr, y_prev, dis_pad, s_prev)


def kernel(edge_index, user_weight, item_weight):
  row_arr = edge_index[0]
  col_arr = edge_index[1]
  x0 = jnp.concatenate([user_weight, item_weight], axis=0)
  x0p = jnp.zeros((NPAD, EMB), jnp.float32)
  x0p = x0p.at[:HALF].set(x0[:HALF]).at[PHALF:PHALF + HALF].set(x0[HALF:])

  dis_pad, y = _deg_dis_y0(col_arr, x0p)
  s = x0p
  for _ in range(N_LAYERS):
    s, y = _layer(row_arr, col_arr, y, dis_pad, s)

  final = jnp.concatenate([s[:HALF], s[PHALF:PHALF + HALF]], axis=0)
  final = final * (1.0 / (N_LAYERS + 1))
  return (final[:N_USERS], final[N_USERS:])
